# Initial kernel scaffold; baseline (speedup 1.0000x reference)
import jax
import jax.numpy as jnp
from jax.experimental import pallas as pl


def kernel(molecular_features, mask, W, b, gamma, beta):
    raise NotImplementedError("write your pallas kernel here")


# speedup vs baseline: 25.8477x; 25.8477x over previous
import functools

import jax
import jax.numpy as jnp
import numpy as np
from jax.experimental import pallas as pl
from jax.experimental.pallas import tpu as pltpu

_STE_SPARSITY = 0.2
_I32_MIN = np.int32(-(2**31))


def _coder_tile(x_ref, wt_ref, b_ref, g_ref, bt_ref, m_ref, o_ref, *, k, apb, nb):
    a = apb * nb
    t = x_ref.shape[0]
    x = x_ref[...]
    wt = wt_ref[...]
    raw_t = jax.lax.dot_general(
        wt, x, (((1,), (1,)), ((), ())),
        preferred_element_type=jnp.float32,
    )
    raw_t = raw_t + b_ref[...]

    g3 = raw_t.reshape(nb, apb, t)
    mu = jnp.mean(g3, axis=1, keepdims=True)
    xc = g3 - mu
    var = jnp.mean(xc * xc, axis=1, keepdims=True)
    inv = jax.lax.rsqrt(var + 1e-5)
    norm_t = (xc * inv).reshape(a, t) * g_ref[...] + bt_ref[...]

    bits = jax.lax.bitcast_convert_type(norm_t, jnp.int32)
    s = bits ^ (jax.lax.shift_right_arithmetic(bits, 31) & np.int32(0x7FFFFFFF))

    p = jnp.zeros((1, t), jnp.int32)
    for bpos in range(31, -1, -1):
        bitv = _I32_MIN if bpos == 31 else np.int32(1 << bpos)
        c = p | bitv
        cs = c ^ _I32_MIN
        cnt = jnp.sum((s >= cs).astype(jnp.int32), axis=0, keepdims=True)
        p = jnp.where(cnt >= k, c, p)
    ps = p ^ _I32_MIN

    bits_mask_t = (s >= ps).astype(jnp.float32)

    rows = jax.lax.broadcasted_iota(jnp.int32, (a, a), 0)
    cols = jax.lax.broadcasted_iota(jnp.int32, (a, a), 1)
    eye = (rows == cols).astype(jnp.float32)
    out = jax.lax.dot_general(
        bits_mask_t, eye, (((0,), (0,)), ((), ())),
        preferred_element_type=jnp.float32,
        precision=jax.lax.Precision.HIGHEST,
    )
    o_ref[...] = out * m_ref[0]


def kernel(molecular_features, mask, W, b, gamma, beta):
    B, N, D = molecular_features.shape
    nb, _, apb = W.shape
    a = nb * apb
    k = max(1, int(a * _STE_SPARSITY))
    bn = B * N

    T = 512
    grid = (bn // T,)

    x2d = molecular_features.reshape(bn, D)
    wt = jnp.transpose(W, (0, 2, 1)).reshape(a, D)
    b_col = b.reshape(a, 1)
    g_col = gamma.reshape(a, 1)
    bt_col = beta.reshape(a, 1)
    m3 = mask.reshape(bn // T, T, 1)

    out = pl.pallas_call(
        functools.partial(_coder_tile, k=k, apb=apb, nb=nb),
        grid=grid,
        in_specs=[
            pl.BlockSpec((T, D), lambda i: (i, 0)),
            pl.BlockSpec((a, D), lambda i: (0, 0)),
            pl.BlockSpec((a, 1), lambda i: (0, 0)),
            pl.BlockSpec((a, 1), lambda i: (0, 0)),
            pl.BlockSpec((a, 1), lambda i: (0, 0)),
            pl.BlockSpec((1, T, 1), lambda i: (i, 0, 0)),
        ],
        out_specs=pl.BlockSpec((T, a), lambda i: (i, 0)),
        out_shape=jax.ShapeDtypeStruct((bn, a), jnp.float32),
        compiler_params=pltpu.CompilerParams(
            dimension_semantics=("arbitrary",),
        ),
    )(x2d, wt, b_col, g_col, bt_col, m3)

    return out.reshape(B, N, a)


# speedup vs baseline: 35.1814x; 1.3611x over previous
import functools

import jax
import jax.numpy as jnp
import numpy as np
from jax.experimental import pallas as pl
from jax.experimental.pallas import tpu as pltpu

_STE_SPARSITY = 0.2
_I16_MIN = np.int16(-(2**15))


def _count_rows(pred):
    v = pred.astype(jnp.int16)
    n = v.shape[0]
    while n > 8:
        n //= 2
        v = v[:n] + v[n:]
    return jnp.sum(v.astype(jnp.int32), axis=0, keepdims=True).astype(jnp.int16)


def _coder_tile(x_ref, wt_ref, b_ref, g_ref, bt_ref, m_ref, eye_ref, o_ref,
                *, k, apb, nb):
    a = apb * nb
    t = x_ref.shape[0]
    x = x_ref[...]
    wt = wt_ref[...]
    raw_t = jax.lax.dot_general(
        wt, x, (((1,), (1,)), ((), ())),
        preferred_element_type=jnp.float32,
    )
    raw_t = raw_t + b_ref[...]

    g3 = raw_t.reshape(nb, apb, t)
    mu = jnp.mean(g3, axis=1, keepdims=True)
    xc = g3 - mu
    var = jnp.mean(xc * xc, axis=1, keepdims=True)
    inv = jax.lax.rsqrt(var + 1e-5)
    norm_t = (xc * inv).reshape(a, t) * g_ref[...] + bt_ref[...]

    bits = jax.lax.bitcast_convert_type(norm_t, jnp.int32)
    s = bits ^ (jax.lax.shift_right_arithmetic(bits, 31) & np.int32(0x7FFFFFFF))

    s_hi = jax.lax.shift_right_arithmetic(s, 16).astype(jnp.int16)
    s_lo = s.astype(jnp.int16) ^ _I16_MIN

    kk = jnp.int16(k)

    p = jnp.zeros((1, t), jnp.int16)
    for bpos in range(15, -1, -1):
        bitv = _I16_MIN if bpos == 15 else np.int16(1 << bpos)
        c = p | bitv
        cs = c ^ _I16_MIN
        cnt = _count_rows(s_hi >= cs)
        p = jnp.where(cnt >= kk, c, p)
    hs = p ^ _I16_MIN

    above = s_hi > hs
    tie = s_hi == hs
    cnt_gt = _count_rows(above)
    need = kk - cnt_gt

    lz = jnp.where(tie, s_lo, _I16_MIN)
    p2 = jnp.zeros((1, t), jnp.int16)
    for bpos in range(15, -1, -1):
        bitv = _I16_MIN if bpos == 15 else np.int16(1 << bpos)
        c = p2 | bitv
        cs = c ^ _I16_MIN
        cnt = _count_rows(lz >= cs)
        p2 = jnp.where(cnt >= need, c, p2)
    p2s = p2 ^ _I16_MIN

    keep = above | (tie & (s_lo >= p2s))
    bits_mask_t = keep.astype(jnp.float32)

    out = jax.lax.dot_general(
        bits_mask_t, eye_ref[...], (((0,), (0,)), ((), ())),
        preferred_element_type=jnp.float32,
    )
    o_ref[...] = out * m_ref[0]


def kernel(molecular_features, mask, W, b, gamma, beta):
    B, N, D = molecular_features.shape
    nb, _, apb = W.shape
    a = nb * apb
    k = max(1, int(a * _STE_SPARSITY))
    bn = B * N

    T = 512
    grid = (bn // T,)

    x2d = molecular_features.reshape(bn, D)
    wt = jnp.transpose(W, (0, 2, 1)).reshape(a, D)
    b_col = b.reshape(a, 1)
    g_col = gamma.reshape(a, 1)
    bt_col = beta.reshape(a, 1)
    m3 = mask.reshape(bn // T, T, 1)
    eye = jnp.eye(a, dtype=jnp.float32)

    out = pl.pallas_call(
        functools.partial(_coder_tile, k=k, apb=apb, nb=nb),
        grid=grid,
        in_specs=[
            pl.BlockSpec((T, D), lambda i: (i, 0)),
            pl.BlockSpec((a, D), lambda i: (0, 0)),
            pl.BlockSpec((a, 1), lambda i: (0, 0)),
            pl.BlockSpec((a, 1), lambda i: (0, 0)),
            pl.BlockSpec((a, 1), lambda i: (0, 0)),
            pl.BlockSpec((1, T, 1), lambda i: (i, 0, 0)),
            pl.BlockSpec((a, a), lambda i: (0, 0)),
        ],
        out_specs=pl.BlockSpec((T, a), lambda i: (i, 0)),
        out_shape=jax.ShapeDtypeStruct((bn, a), jnp.float32),
        compiler_params=pltpu.CompilerParams(
            dimension_semantics=("arbitrary",),
        ),
    )(x2d, wt, b_col, g_col, bt_col, m3, eye)

    return out.reshape(B, N, a)


# speedup vs baseline: 57.9775x; 1.6480x over previous
import functools

import jax
import jax.numpy as jnp
import numpy as np
from jax.experimental import pallas as pl
from jax.experimental.pallas import tpu as pltpu

_STE_SPARSITY = 0.2
_I16_MIN = np.int16(-(2**15))


def _count_rows(pred):
    v = pred.astype(jnp.int16)
    n = v.shape[0]
    while n > 8:
        n //= 2
        v = v[:n] + v[n:]
    return jnp.sum(v.astype(jnp.int32), axis=0, keepdims=True).astype(jnp.int16)


def _coder_tile(x_ref, wt_ref, b_ref, g_ref, bt_ref, m_ref, eye_ref, o_ref,
                *, k, apb, nb):
    a = apb * nb
    t = x_ref.shape[0]
    x = x_ref[...]
    wt = wt_ref[...]
    raw_t = jax.lax.dot_general(
        wt, x, (((1,), (1,)), ((), ())),
        preferred_element_type=jnp.float32,
    )
    raw_t = raw_t + b_ref[...]

    g3 = raw_t.reshape(nb, apb, t)
    mu = jnp.mean(g3, axis=1, keepdims=True)
    xc = g3 - mu
    var = jnp.mean(xc * xc, axis=1, keepdims=True)
    inv = jax.lax.rsqrt(var + 1e-5)
    norm_t = (xc * inv).reshape(a, t) * g_ref[...] + bt_ref[...]

    bits = jax.lax.bitcast_convert_type(norm_t, jnp.int32)
    s = bits ^ (jax.lax.shift_right_arithmetic(bits, 31) & np.int32(0x7FFFFFFF))

    s_hi = jax.lax.shift_right_arithmetic(s, 16).astype(jnp.int16)
    s_lo = s.astype(jnp.int16) ^ _I16_MIN

    kk = jnp.int16(k)

    p = jnp.zeros((1, t), jnp.int16)
    for bpos in range(15, -1, -1):
        bitv = _I16_MIN if bpos == 15 else np.int16(1 << bpos)
        c = p | bitv
        cs = c ^ _I16_MIN
        cnt = _count_rows(s_hi >= cs)
        p = jnp.where(cnt >= kk, c, p)
    hs = p ^ _I16_MIN

    above = s_hi > hs
    tie = s_hi == hs
    cnt_gt = _count_rows(above)
    need = kk - cnt_gt

    lz = jnp.where(tie, s_lo, _I16_MIN)
    p2 = jnp.zeros((1, t), jnp.int16)
    for bpos in range(15, -1, -1):
        bitv = _I16_MIN if bpos == 15 else np.int16(1 << bpos)
        c = p2 | bitv
        cs = c ^ _I16_MIN
        cnt = _count_rows(lz >= cs)
        p2 = jnp.where(cnt >= need, c, p2)
    p2s = p2 ^ _I16_MIN

    keep = above | (tie & (s_lo >= p2s))
    bits_mask_t = keep.astype(jnp.float32)

    out = jax.lax.dot_general(
        bits_mask_t, eye_ref[...], (((0,), (0,)), ((), ())),
        preferred_element_type=jnp.float32,
    )
    o_ref[...] = out * m_ref[0]


def kernel(molecular_features, mask, W, b, gamma, beta):
    B, N, D = molecular_features.shape
    nb, _, apb = W.shape
    a = nb * apb
    k = max(1, int(a * _STE_SPARSITY))
    bn = B * N

    T = 4096
    grid = (bn // T,)

    x2d = molecular_features.reshape(bn, D)
    wt = jnp.transpose(W, (0, 2, 1)).reshape(a, D)
    b_col = b.reshape(a, 1)
    g_col = gamma.reshape(a, 1)
    bt_col = beta.reshape(a, 1)
    m3 = mask.reshape(bn // T, T, 1)
    eye = jnp.eye(a, dtype=jnp.float32)

    out = pl.pallas_call(
        functools.partial(_coder_tile, k=k, apb=apb, nb=nb),
        grid=grid,
        in_specs=[
            pl.BlockSpec((T, D), lambda i: (i, 0)),
            pl.BlockSpec((a, D), lambda i: (0, 0)),
            pl.BlockSpec((a, 1), lambda i: (0, 0)),
            pl.BlockSpec((a, 1), lambda i: (0, 0)),
            pl.BlockSpec((a, 1), lambda i: (0, 0)),
            pl.BlockSpec((1, T, 1), lambda i: (i, 0, 0)),
            pl.BlockSpec((a, a), lambda i: (0, 0)),
        ],
        out_specs=pl.BlockSpec((T, a), lambda i: (i, 0)),
        out_shape=jax.ShapeDtypeStruct((bn, a), jnp.float32),
        compiler_params=pltpu.CompilerParams(
            dimension_semantics=("arbitrary",),
        ),
    )(x2d, wt, b_col, g_col, bt_col, m3, eye)

    return out.reshape(B, N, a)
